# baseline (speedup 1.0000x reference)
import functools

import jax
import jax.numpy as jnp
from jax import lax
from jax.experimental import pallas as pl
from jax.experimental.pallas import tpu as pltpu
from jax.experimental.pallas import tpu_sc as plsc

_B, _S, _D = 4, 4096, 1024
_SSC = 512
_S0 = _S - _SSC
_BS = 1792

_NC, _NS = 2, 16
_NW = _NC * _NS
_WPB = _NW // _B
_ROWS_W = _SSC // _WPB
_CH = 16
_NCHUNK = _ROWS_W // _CH
_NBUF = 3
_UNROLL = 16
_L = 16
_CD = _CH * _D


def _add_body(x_ref, pe_ref, o_ref):
    o_ref[...] = x_ref[...] + pe_ref[...][None]


def _tc_add(x, pe_table):
    grid = (_S0 // _BS, _B)
    return pl.pallas_call(
        _add_body,
        grid=grid,
        in_specs=[
            pl.BlockSpec((1, _BS, _D), lambda s, b: (b, s, 0)),
            pl.BlockSpec((_BS, _D), lambda s, b: (s, 0)),
        ],
        out_specs=pl.BlockSpec((1, _BS, _D), lambda s, b: (b, s, 0)),
        out_shape=jax.ShapeDtypeStruct((_B, _S, _D), x.dtype),
        compiler_params=pltpu.CompilerParams(
            dimension_semantics=("arbitrary", "arbitrary"),
        ),
    )(x, pe_table)


def _sc_body(x_hbm, pe_hbm, o_hbm, xbufs, pebufs, xsems, psems, osems):
    c = lax.axis_index("c")
    s = lax.axis_index("s")
    wid = s * _NC + c
    b = wid // _WPB
    s_loc = (wid % _WPB) * _ROWS_W
    x_base = b * _S + _S0 + s_loc
    pe_base = _S0 + s_loc
    o_base = b * _SSC + s_loc

    def in_copies(i):
        sl = i % _NBUF
        return (
            pltpu.make_async_copy(
                x_hbm.at[pl.ds((x_base + i * _CH) * _D, _CD)], xbufs[sl], xsems[sl]),
            pltpu.make_async_copy(
                pe_hbm.at[pl.ds((pe_base + i * _CH) * _D, _CD)], pebufs[sl], psems[sl]),
        )

    def out_copy(i):
        sl = i % _NBUF
        return pltpu.make_async_copy(
            xbufs[sl], o_hbm.at[pl.ds((o_base + i * _CH) * _D, _CD)], osems[sl])

    for cp in in_copies(0) + in_copies(1):
        cp.start()

    for i in range(_NCHUNK):
        sl = i % _NBUF
        for cp in in_copies(i):
            cp.wait()
        if i + 2 < _NCHUNK:
            if i >= 1:
                out_copy(i - 1).wait()
            for cp in in_copies(i + 2):
                cp.start()

        xv = xbufs[sl]
        pv = pebufs[sl]

        def vec(j, carry):
            base = j * (_L * _UNROLL)
            for u in range(_UNROLL):
                ds = pl.ds(base + u * _L, _L)
                xv[ds] = xv[ds] + pv[ds]
            return carry

        lax.fori_loop(0, _CD // (_L * _UNROLL), vec, 0)
        out_copy(i).start()

    for i in range(max(0, _NCHUNK - 3), _NCHUNK):
        out_copy(i).wait()


@functools.partial(
    pl.kernel,
    out_type=jax.ShapeDtypeStruct((_B * _SSC * _D,), jnp.float32),
    mesh=plsc.VectorSubcoreMesh(core_axis_name="c", subcore_axis_name="s"),
    scratch_types=[
        [pltpu.VMEM((_CD,), jnp.float32)] * _NBUF,
        [pltpu.VMEM((_CD,), jnp.float32)] * _NBUF,
        [pltpu.SemaphoreType.DMA] * _NBUF,
        [pltpu.SemaphoreType.DMA] * _NBUF,
        [pltpu.SemaphoreType.DMA] * _NBUF,
    ],
)
def _sc_kernel(x_hbm, pe_hbm, o_hbm, xbufs, pebufs, xsems, psems, osems):
    _sc_body(x_hbm, pe_hbm, o_hbm, xbufs, pebufs, xsems, psems, osems)


def kernel(x, pe_table):
    B, S, D = x.shape
    tc_out = _tc_add(x, pe_table)
    sc_out = _sc_kernel(x.reshape(-1), pe_table[:S].reshape(-1))
    return lax.dynamic_update_slice(
        tc_out, sc_out.reshape(B, _SSC, D), (0, _S0, 0))


# speedup vs baseline: 3.1894x; 3.1894x over previous
import jax
import jax.numpy as jnp
from jax.experimental import pallas as pl
from jax.experimental.pallas import tpu as pltpu

_BS = 2048


def _add_body(x_ref, pe_ref, o_ref):
    o_ref[...] = x_ref[...] + pe_ref[...][None]


def kernel(x, pe_table):
    B, S, D = x.shape
    grid = (S // _BS, B)
    return pl.pallas_call(
        _add_body,
        grid=grid,
        in_specs=[
            pl.BlockSpec((1, _BS, D), lambda s, b: (b, s, 0)),
            pl.BlockSpec((_BS, D), lambda s, b: (s, 0)),
        ],
        out_specs=pl.BlockSpec((1, _BS, D), lambda s, b: (b, s, 0)),
        out_shape=jax.ShapeDtypeStruct((B, S, D), x.dtype),
        compiler_params=pltpu.CompilerParams(
            dimension_semantics=("arbitrary", "arbitrary"),
        ),
    )(x, pe_table)


# speedup vs baseline: 3.1962x; 1.0021x over previous
import jax
import jax.numpy as jnp
from jax.experimental import pallas as pl
from jax.experimental.pallas import tpu as pltpu

_BS = 2048


def _add_body(x_ref, pe_ref, o_ref):
    o_ref[...] = x_ref[...] + pe_ref[...][None]


def kernel(x, pe_table):
    B, S, D = x.shape
    grid = (S // _BS, B)
    return pl.pallas_call(
        _add_body,
        grid=grid,
        in_specs=[
            pl.BlockSpec((1, _BS, D), lambda s, b: (b, s, 0)),
            pl.BlockSpec((_BS, D), lambda s, b: (s, 0)),
        ],
        out_specs=pl.BlockSpec((1, _BS, D), lambda s, b: (b, s, 0)),
        out_shape=jax.ShapeDtypeStruct((B, S, D), x.dtype),
        compiler_params=pltpu.CompilerParams(
            dimension_semantics=("parallel", "parallel"),
        ),
    )(x, pe_table)
